# baseline (speedup 1.0000x reference)
import functools

import jax
import jax.numpy as jnp
from jax import lax
from jax.experimental import pallas as pl
from jax.experimental.pallas import tpu as pltpu
from jax.experimental.pallas import tpu_sc as plsc

N = 10000
E = 320000
D = 128

NC = 2
NS = 16
NW = NC * NS
C = 64
ROWS_PER_W = 160
NCHUNK = 4
KCH = ROWS_PER_W // NCHUNK
EPAD = NW * ROWS_PER_W * C
NP = 10112
RPS = NP // NS
NPD = 10240
DPS = NPD // NS

_mesh = plsc.VectorSubcoreMesh(core_axis_name="c", subcore_axis_name="s")


@functools.partial(
    pl.kernel,
    out_type=jax.ShapeDtypeStruct((NC * NPD,), jnp.float32),
    mesh=_mesh,
    scratch_types=[
        pltpu.VMEM_SHARED((NPD,), jnp.float32),
        pltpu.VMEM((KCH, C), jnp.int32),
        pltpu.VMEM((KCH, C), jnp.float32),
        pltpu.SemaphoreType.DMA,
    ],
)
def _deg_kernel(dst_hbm, w_hbm, zerod_hbm, out_hbm, dacc, dstv, wv, sem):
    cid = lax.axis_index("c")
    sid = lax.axis_index("s")
    wid = sid * NC + cid
    pltpu.sync_copy(zerod_hbm, dacc.at[pl.ds(sid * DPS, DPS)])
    plsc.subcore_barrier()

    def chunk(q, carry):
        pltpu.sync_copy(dst_hbm.at[wid, q], dstv)
        pltpu.sync_copy(w_hbm.at[wid, q], wv)

        def body(j, c2):
            pltpu.async_copy(wv.at[j], dacc.at[dstv.at[j]], sem, add=True)
            return c2

        lax.fori_loop(0, KCH, body, carry)

        def drain(j, c2):
            pltpu.make_async_copy(wv.at[j], dacc.at[dstv.at[j]], sem).wait()
            return c2

        return lax.fori_loop(0, KCH, drain, carry)

    lax.fori_loop(0, NCHUNK, chunk, 0)
    plsc.subcore_barrier()
    pltpu.sync_copy(dacc.at[pl.ds(sid * DPS, DPS)],
                    out_hbm.at[pl.ds(cid * NPD + sid * DPS, DPS)])


@functools.partial(
    pl.kernel,
    out_type=jax.ShapeDtypeStruct((NC * NP, D), jnp.float32),
    mesh=_mesh,
    scratch_types=[
        pltpu.VMEM_SHARED((NP, D), jnp.float32),
        pltpu.VMEM((KCH, C), jnp.int32),
        pltpu.VMEM((KCH, C), jnp.int32),
        pltpu.VMEM((KCH, C), jnp.float32),
        pltpu.VMEM((C, D), jnp.float32),
        pltpu.VMEM((C, D), jnp.float32),
        pltpu.VMEM((C, D), jnp.float32),
        pltpu.VMEM((C, D), jnp.float32),
        pltpu.SemaphoreType.DMA,
        pltpu.SemaphoreType.DMA,
    ],
)
def _msg_kernel(h_hbm, src_hbm, dst_hbm, w_hbm, zeros_hbm, out_hbm,
                acc, srcv, dstv, wv, rb0, rb1, rb2, rb3, sem_g, sem_s):
    cid = lax.axis_index("c")
    sid = lax.axis_index("s")
    wid = sid * NC + cid
    pltpu.sync_copy(zeros_hbm, acc.at[pl.ds(sid * RPS, RPS)])
    plsc.subcore_barrier()
    rbs = (rb0, rb1, rb2, rb3)

    def _scale(j, rb):
        @plsc.parallel_loop(0, C // 16)
        def grp(g):
            wvec = wv[j, pl.ds(g * 16, 16)]
            for l in range(16):
                i = g * 16 + l
                s = wvec[l]
                for k in range(D // 16):
                    sl = pl.ds(k * 16, 16)
                    rb[i, sl] = rb[i, sl] * s

    def chunk(q, carry):
        pltpu.sync_copy(src_hbm.at[wid, q], srcv)
        pltpu.sync_copy(dst_hbm.at[wid, q], dstv)
        pltpu.sync_copy(w_hbm.at[wid, q], wv)
        pltpu.async_copy(h_hbm.at[srcv.at[0]], rbs[0], sem_g)
        pltpu.async_copy(h_hbm.at[srcv.at[1]], rbs[1], sem_g)

        def quad(t, c2):
            for u in range(4):
                j = t * 4 + u
                rb = rbs[u]
                nxt = rbs[(u + 2) % 4]
                if u < 2:
                    @pl.when(t == 0)
                    def _():
                        pltpu.async_copy(h_hbm.at[srcv.at[u + 2]], nxt, sem_g)

                    @pl.when(t >= 1)
                    def _():
                        pltpu.make_async_copy(
                            nxt, acc.at[dstv.at[j - 2]], sem_s).wait()
                        pltpu.async_copy(h_hbm.at[srcv.at[j + 2]], nxt, sem_g)
                else:
                    pltpu.make_async_copy(
                        nxt, acc.at[dstv.at[j - 2]], sem_s).wait()

                    @pl.when(j + 2 <= KCH - 1)
                    def _():
                        pltpu.async_copy(h_hbm.at[srcv.at[j + 2]], nxt, sem_g)
                pltpu.make_async_copy(h_hbm.at[srcv.at[j]], rb, sem_g).wait()
                _scale(j, rb)
                pltpu.async_copy(rb, acc.at[dstv.at[j]], sem_s, add=True)
            return c2

        lax.fori_loop(0, KCH // 4, quad, carry)
        for u in range(2, 4):
            pltpu.make_async_copy(
                rbs[u], acc.at[dstv.at[KCH - 4 + u]], sem_s).wait()
        return carry

    lax.fori_loop(0, NCHUNK, chunk, 0)
    plsc.subcore_barrier()
    pltpu.sync_copy(acc.at[pl.ds(sid * RPS, RPS)],
                    out_hbm.at[pl.ds(cid * NP + sid * RPS, RPS)])


BN = 1000


def _kb_body(x_ref, w1_ref, dp_ref, dq_ref, h_ref, dis_ref):
    deg = dp_ref[0] + dq_ref[0] + 1.0
    dis = lax.rsqrt(deg)
    h = jnp.dot(x_ref[...], w1_ref[...], preferred_element_type=jnp.float32)
    h_ref[...] = h * dis
    dis_ref[...] = dis


def _kd_body(sa_ref, sb_ref, h1_ref, dis_ref, b1_ref, w2_ref, h2_ref):
    dis = dis_ref[...]
    t = (sa_ref[0] + sb_ref[0] + h1_ref[...]) * dis + b1_ref[...]
    x1 = jnp.maximum(t, 0.0)
    h2_ref[...] = jnp.dot(x1, w2_ref[...],
                          preferred_element_type=jnp.float32) * dis


def _kf_body(sa_ref, sb_ref, h2_ref, dis_ref, b2_ref, o_ref):
    o_ref[...] = ((sa_ref[0] + sb_ref[0] + h2_ref[...]) * dis_ref[...]
                  + b2_ref[...])


def _row_spec(i):
    return (i, 0)


def _rep_spec(i):
    return (0, 0)


def _core0_spec(i):
    return (0, i, 0)


def _core1_spec(i):
    return (1, i, 0)


_kb = pl.pallas_call(
    _kb_body,
    grid=(N // BN,),
    in_specs=[
        pl.BlockSpec((BN, D), _row_spec),
        pl.BlockSpec((D, D), _rep_spec),
        pl.BlockSpec((1, BN, 1), _core0_spec),
        pl.BlockSpec((1, BN, 1), _core1_spec),
    ],
    out_specs=[
        pl.BlockSpec((BN, D), _row_spec),
        pl.BlockSpec((BN, 1), _row_spec),
    ],
    out_shape=[
        jax.ShapeDtypeStruct((N, D), jnp.float32),
        jax.ShapeDtypeStruct((N, 1), jnp.float32),
    ],
)

_kd = pl.pallas_call(
    _kd_body,
    grid=(N // BN,),
    in_specs=[
        pl.BlockSpec((1, BN, D), _core0_spec),
        pl.BlockSpec((1, BN, D), _core1_spec),
        pl.BlockSpec((BN, D), _row_spec),
        pl.BlockSpec((BN, 1), _row_spec),
        pl.BlockSpec((1, D), _rep_spec),
        pl.BlockSpec((D, D), _rep_spec),
    ],
    out_specs=pl.BlockSpec((BN, D), _row_spec),
    out_shape=jax.ShapeDtypeStruct((N, D), jnp.float32),
)

_kf = pl.pallas_call(
    _kf_body,
    grid=(N // BN,),
    in_specs=[
        pl.BlockSpec((1, BN, D), _core0_spec),
        pl.BlockSpec((1, BN, D), _core1_spec),
        pl.BlockSpec((BN, D), _row_spec),
        pl.BlockSpec((BN, 1), _row_spec),
        pl.BlockSpec((1, D), _rep_spec),
    ],
    out_specs=pl.BlockSpec((BN, D), _row_spec),
    out_shape=jax.ShapeDtypeStruct((N, D), jnp.float32),
)


def kernel(node_matrix, edge_index, edge_weights, W1, b1, W2, b2):
    pad_idx = jnp.arange(EPAD - E, dtype=jnp.int32) % N
    src3 = jnp.concatenate(
        [edge_index[0].astype(jnp.int32), pad_idx]).reshape(NW, NCHUNK, KCH, C)
    dst3 = jnp.concatenate(
        [edge_index[1].astype(jnp.int32), pad_idx]).reshape(NW, NCHUNK, KCH, C)
    w3 = jnp.concatenate(
        [edge_weights, jnp.zeros((EPAD - E,), jnp.float32)]
    ).reshape(NW, NCHUNK, KCH, C)

    zerod = jnp.zeros((DPS,), jnp.float32)
    zeros = jnp.zeros((RPS, D), jnp.float32)

    deg3 = _deg_kernel(dst3, w3, zerod).reshape(NC, NPD, 1)
    h1, dis = _kb(node_matrix, W1, deg3, deg3)
    s1 = _msg_kernel(h1, src3, dst3, w3, zeros).reshape(NC, NP, D)
    h2 = _kd(s1, s1, h1, dis, b1.reshape(1, D), W2)
    s2 = _msg_kernel(h2, src3, dst3, w3, zeros).reshape(NC, NP, D)
    out = _kf(s2, s2, h2, dis, b2.reshape(1, D))
    return out


# speedup vs baseline: 1.0667x; 1.0667x over previous
import functools

import jax
import jax.numpy as jnp
from jax import lax
from jax.experimental import pallas as pl
from jax.experimental.pallas import tpu as pltpu
from jax.experimental.pallas import tpu_sc as plsc

N = 10000
E = 320000
D = 128

NC = 2
NS = 16
NW = NC * NS
C = 64
ROWS_PER_W = 160
NCHUNK = 4
KCH = ROWS_PER_W // NCHUNK
EPAD = NW * ROWS_PER_W * C
NP = 10112
RPS = NP // NS
NPD = 10240
DPS = NPD // NS

_mesh = plsc.VectorSubcoreMesh(core_axis_name="c", subcore_axis_name="s")


@functools.partial(
    pl.kernel,
    out_type=jax.ShapeDtypeStruct((NC * NPD,), jnp.float32),
    mesh=_mesh,
    scratch_types=[
        pltpu.VMEM_SHARED((NPD,), jnp.float32),
        pltpu.VMEM((KCH, C), jnp.int32),
        pltpu.VMEM((KCH, C), jnp.float32),
        pltpu.SemaphoreType.DMA,
    ],
)
def _deg_kernel(dst_hbm, w_hbm, zerod_hbm, out_hbm, dacc, dstv, wv, sem):
    cid = lax.axis_index("c")
    sid = lax.axis_index("s")
    wid = sid * NC + cid
    pltpu.sync_copy(zerod_hbm, dacc.at[pl.ds(sid * DPS, DPS)])
    plsc.subcore_barrier()

    def chunk(q, carry):
        pltpu.sync_copy(dst_hbm.at[wid, q], dstv)
        pltpu.sync_copy(w_hbm.at[wid, q], wv)

        def body(j, c2):
            pltpu.async_copy(wv.at[j], dacc.at[dstv.at[j]], sem, add=True)
            return c2

        lax.fori_loop(0, KCH, body, carry)

        def drain(j, c2):
            pltpu.make_async_copy(wv.at[j], dacc.at[dstv.at[j]], sem).wait()
            return c2

        return lax.fori_loop(0, KCH, drain, carry)

    lax.fori_loop(0, NCHUNK, chunk, 0)
    plsc.subcore_barrier()
    pltpu.sync_copy(dacc.at[pl.ds(sid * DPS, DPS)],
                    out_hbm.at[pl.ds(cid * NPD + sid * DPS, DPS)])


@functools.partial(
    pl.kernel,
    out_type=jax.ShapeDtypeStruct((NC * NP, D), jnp.float32),
    mesh=_mesh,
    scratch_types=[
        pltpu.VMEM_SHARED((NP, D), jnp.float32),
        pltpu.VMEM((KCH, C), jnp.int32),
        pltpu.VMEM((KCH, C), jnp.int32),
        pltpu.VMEM((KCH, C), jnp.float32),
        pltpu.VMEM((C, D), jnp.float32),
        pltpu.VMEM((C, D), jnp.float32),
        pltpu.VMEM((C, D), jnp.float32),
        pltpu.VMEM((C, D), jnp.float32),
        pltpu.SemaphoreType.DMA,
        pltpu.SemaphoreType.DMA,
    ],
)
def _msg_kernel(h_hbm, src_hbm, dst_hbm, w_hbm, zeros_hbm, out_hbm,
                acc, srcv, dstv, wv, rb0, rb1, rb2, rb3, sem_g, sem_s):
    cid = lax.axis_index("c")
    sid = lax.axis_index("s")
    wid = sid * NC + cid
    pltpu.sync_copy(zeros_hbm, acc.at[pl.ds(sid * RPS, RPS)])
    plsc.subcore_barrier()
    rbs = (rb0, rb1, rb2, rb3)

    def _scale(j, rb):
        def grp(g, c3):
            wvec = wv[j, pl.ds(g * 16, 16)]
            for l in range(16):
                i = g * 16 + l
                s = wvec[l]
                for k in range(D // 16):
                    sl = pl.ds(k * 16, 16)
                    rb[i, sl] = rb[i, sl] * s
            return c3

        lax.fori_loop(0, C // 16, grp, 0)

    def chunk(q, carry):
        pltpu.sync_copy(src_hbm.at[wid, q], srcv)
        pltpu.sync_copy(dst_hbm.at[wid, q], dstv)
        pltpu.sync_copy(w_hbm.at[wid, q], wv)
        pltpu.async_copy(h_hbm.at[srcv.at[0]], rbs[0], sem_g)
        pltpu.async_copy(h_hbm.at[srcv.at[1]], rbs[1], sem_g)

        def quad(t, c2):
            for u in range(4):
                j = t * 4 + u
                rb = rbs[u]
                nxt = rbs[(u + 2) % 4]
                if u < 2:
                    @pl.when(t == 0)
                    def _():
                        pltpu.async_copy(h_hbm.at[srcv.at[u + 2]], nxt, sem_g)

                    @pl.when(t >= 1)
                    def _():
                        pltpu.make_async_copy(
                            nxt, acc.at[dstv.at[j - 2]], sem_s).wait()
                        pltpu.async_copy(h_hbm.at[srcv.at[j + 2]], nxt, sem_g)
                else:
                    pltpu.make_async_copy(
                        nxt, acc.at[dstv.at[j - 2]], sem_s).wait()

                    @pl.when(j + 2 <= KCH - 1)
                    def _():
                        pltpu.async_copy(h_hbm.at[srcv.at[j + 2]], nxt, sem_g)
                pltpu.make_async_copy(h_hbm.at[srcv.at[j]], rb, sem_g).wait()
                _scale(j, rb)
                pltpu.async_copy(rb, acc.at[dstv.at[j]], sem_s, add=True)
            return c2

        lax.fori_loop(0, KCH // 4, quad, carry)
        for u in range(2, 4):
            pltpu.make_async_copy(
                rbs[u], acc.at[dstv.at[KCH - 4 + u]], sem_s).wait()
        return carry

    lax.fori_loop(0, NCHUNK, chunk, 0)
    plsc.subcore_barrier()
    pltpu.sync_copy(acc.at[pl.ds(sid * RPS, RPS)],
                    out_hbm.at[pl.ds(cid * NP + sid * RPS, RPS)])


BN = 1000


def _kb_body(x_ref, w1_ref, dp_ref, dq_ref, h_ref, dis_ref):
    deg = dp_ref[0] + dq_ref[0] + 1.0
    dis = lax.rsqrt(deg)
    h = jnp.dot(x_ref[...], w1_ref[...], preferred_element_type=jnp.float32)
    h_ref[...] = h * dis
    dis_ref[...] = dis


def _kd_body(sa_ref, sb_ref, h1_ref, dis_ref, b1_ref, w2_ref, h2_ref):
    dis = dis_ref[...]
    t = (sa_ref[0] + sb_ref[0] + h1_ref[...]) * dis + b1_ref[...]
    x1 = jnp.maximum(t, 0.0)
    h2_ref[...] = jnp.dot(x1, w2_ref[...],
                          preferred_element_type=jnp.float32) * dis


def _kf_body(sa_ref, sb_ref, h2_ref, dis_ref, b2_ref, o_ref):
    o_ref[...] = ((sa_ref[0] + sb_ref[0] + h2_ref[...]) * dis_ref[...]
                  + b2_ref[...])


def _row_spec(i):
    return (i, 0)


def _rep_spec(i):
    return (0, 0)


def _core0_spec(i):
    return (0, i, 0)


def _core1_spec(i):
    return (1, i, 0)


_kb = pl.pallas_call(
    _kb_body,
    grid=(N // BN,),
    in_specs=[
        pl.BlockSpec((BN, D), _row_spec),
        pl.BlockSpec((D, D), _rep_spec),
        pl.BlockSpec((1, BN, 1), _core0_spec),
        pl.BlockSpec((1, BN, 1), _core1_spec),
    ],
    out_specs=[
        pl.BlockSpec((BN, D), _row_spec),
        pl.BlockSpec((BN, 1), _row_spec),
    ],
    out_shape=[
        jax.ShapeDtypeStruct((N, D), jnp.float32),
        jax.ShapeDtypeStruct((N, 1), jnp.float32),
    ],
)

_kd = pl.pallas_call(
    _kd_body,
    grid=(N // BN,),
    in_specs=[
        pl.BlockSpec((1, BN, D), _core0_spec),
        pl.BlockSpec((1, BN, D), _core1_spec),
        pl.BlockSpec((BN, D), _row_spec),
        pl.BlockSpec((BN, 1), _row_spec),
        pl.BlockSpec((1, D), _rep_spec),
        pl.BlockSpec((D, D), _rep_spec),
    ],
    out_specs=pl.BlockSpec((BN, D), _row_spec),
    out_shape=jax.ShapeDtypeStruct((N, D), jnp.float32),
)

_kf = pl.pallas_call(
    _kf_body,
    grid=(N // BN,),
    in_specs=[
        pl.BlockSpec((1, BN, D), _core0_spec),
        pl.BlockSpec((1, BN, D), _core1_spec),
        pl.BlockSpec((BN, D), _row_spec),
        pl.BlockSpec((BN, 1), _row_spec),
        pl.BlockSpec((1, D), _rep_spec),
    ],
    out_specs=pl.BlockSpec((BN, D), _row_spec),
    out_shape=jax.ShapeDtypeStruct((N, D), jnp.float32),
)


def kernel(node_matrix, edge_index, edge_weights, W1, b1, W2, b2):
    pad_idx = jnp.arange(EPAD - E, dtype=jnp.int32) % N
    src3 = jnp.concatenate(
        [edge_index[0].astype(jnp.int32), pad_idx]).reshape(NW, NCHUNK, KCH, C)
    dst3 = jnp.concatenate(
        [edge_index[1].astype(jnp.int32), pad_idx]).reshape(NW, NCHUNK, KCH, C)
    w3 = jnp.concatenate(
        [edge_weights, jnp.zeros((EPAD - E,), jnp.float32)]
    ).reshape(NW, NCHUNK, KCH, C)

    zerod = jnp.zeros((DPS,), jnp.float32)
    zeros = jnp.zeros((RPS, D), jnp.float32)

    deg3 = _deg_kernel(dst3, w3, zerod).reshape(NC, NPD, 1)
    h1, dis = _kb(node_matrix, W1, deg3, deg3)
    s1 = _msg_kernel(h1, src3, dst3, w3, zeros).reshape(NC, NP, D)
    h2 = _kd(s1, s1, h1, dis, b1.reshape(1, D), W2)
    s2 = _msg_kernel(h2, src3, dst3, w3, zeros).reshape(NC, NP, D)
    out = _kf(s2, s2, h2, dis, b2.reshape(1, D))
    return out
